# Initial kernel scaffold; baseline (speedup 1.0000x reference)
import functools

import jax
import jax.numpy as jnp
from jax import lax
from jax.experimental import pallas as pl
from jax.experimental.pallas import tpu as pltpu
from jax.experimental.pallas import tpu_sc as plsc

N_NODES = 10000
N_EDGES = 320000
D_FEAT = 128
CHUNK = 80
GROUPS = CHUNK // 16


def _norms_body(h_ref, out_ref):
    h = h_ref[...]
    out_ref[...] = jnp.sqrt(jnp.sum(h * h, axis=1))


def _node_norms(h):
    return pl.pallas_call(
        _norms_body,
        out_shape=jax.ShapeDtypeStruct((h.shape[0],), jnp.float32),
    )(h)


@functools.cache
def _make_edge_kernel():
    info = plsc.get_sparse_core_info()
    num_cores = info.num_cores
    nw = num_cores * info.num_subcores
    e_per_w = N_EDGES // nw
    n_chunks = e_per_w // CHUNK

    mesh = plsc.VectorSubcoreMesh(core_axis_name="c", subcore_axis_name="s")

    @functools.partial(
        pl.kernel,
        mesh=mesh,
        out_type=jax.ShapeDtypeStruct((N_EDGES,), jnp.float32),
        scratch_types=[
            pltpu.VMEM((N_NODES,), jnp.float32),
            pltpu.VMEM((CHUNK,), jnp.int32),
            pltpu.VMEM((CHUNK,), jnp.int32),
            pltpu.VMEM((CHUNK, D_FEAT), jnp.float32),
            pltpu.VMEM((CHUNK, D_FEAT), jnp.float32),
            pltpu.VMEM((CHUNK,), jnp.float32),
            pltpu.SemaphoreType.DMA,
            pltpu.SemaphoreType.DMA,
        ],
    )
    def edge_kernel(h_hbm, src_hbm, dst_hbm, norms_hbm, out_hbm,
                    norms_v, sids_v, dids_v, srows_v, drows_v, outc_v,
                    sem_a, sem_b):
        wid = lax.axis_index("s") * num_cores + lax.axis_index("c")
        wbase = wid * e_per_w
        pltpu.sync_copy(norms_hbm, norms_v)

        def chunk_body(c, carry):
            cbase = wbase + c * CHUNK
            pltpu.sync_copy(src_hbm.at[pl.ds(cbase, CHUNK)], sids_v)
            pltpu.sync_copy(dst_hbm.at[pl.ds(cbase, CHUNK)], dids_v)
            cp1 = pltpu.async_copy(h_hbm.at[sids_v], srows_v, sem_a)
            cp2 = pltpu.async_copy(h_hbm.at[dids_v], drows_v, sem_b)
            cp1.wait()
            cp2.wait()

            def group_body(g, gcarry):
                rows = g * 16 + lax.iota(jnp.int32, 16)

                def dot_body(dd, acc):
                    col = jnp.broadcast_to(dd, (16,))
                    s = plsc.load_gather(srows_v, [rows, col])
                    t = plsc.load_gather(drows_v, [rows, col])
                    return acc + s * t

                num = lax.fori_loop(0, D_FEAT, dot_body,
                                    jnp.zeros((16,), jnp.float32), unroll=8)
                sid = sids_v[pl.ds(g * 16, 16)]
                did = dids_v[pl.ds(g * 16, 16)]
                ns = plsc.load_gather(norms_v, [sid])
                nd = plsc.load_gather(norms_v, [did])
                denom = jnp.maximum(ns * nd, jnp.float32(1e-6))
                outc_v[pl.ds(g * 16, 16)] = num / denom
                return gcarry

            lax.fori_loop(0, GROUPS, group_body, 0)
            pltpu.sync_copy(outc_v, out_hbm.at[pl.ds(cbase, CHUNK)])
            return carry

        lax.fori_loop(0, n_chunks, chunk_body, 0)

    return edge_kernel


def kernel(h, edge_index):
    h = h.astype(jnp.float32)
    ei = edge_index.astype(jnp.int32)
    src = ei[0]
    dst = ei[1]
    norms = _node_norms(h)
    return _make_edge_kernel()(h, src, dst, norms)


# speedup vs baseline: 1.1482x; 1.1482x over previous
import functools

import jax
import jax.numpy as jnp
from jax import lax
from jax.experimental import pallas as pl
from jax.experimental.pallas import tpu as pltpu
from jax.experimental.pallas import tpu_sc as plsc

N_NODES = 10000
N_EDGES = 320000
D_FEAT = 128
CHUNK = 80
GROUPS = CHUNK // 16


def _norms_body(h_ref, out_ref):
    h = h_ref[...]
    out_ref[...] = jnp.sqrt(jnp.sum(h * h, axis=1))


def _node_norms(h):
    return pl.pallas_call(
        _norms_body,
        out_shape=jax.ShapeDtypeStruct((h.shape[0],), jnp.float32),
    )(h)


@functools.cache
def _make_edge_kernel():
    info = plsc.get_sparse_core_info()
    num_cores = info.num_cores
    nw = num_cores * info.num_subcores
    e_per_w = N_EDGES // nw
    n_chunks = e_per_w // CHUNK

    mesh = plsc.VectorSubcoreMesh(core_axis_name="c", subcore_axis_name="s")

    @functools.partial(
        pl.kernel,
        mesh=mesh,
        compiler_params=pltpu.CompilerParams(needs_layout_passes=False),
        out_type=jax.ShapeDtypeStruct((N_EDGES,), jnp.float32),
        scratch_types=[
            pltpu.VMEM((N_NODES,), jnp.float32),
            pltpu.VMEM((CHUNK,), jnp.int32),
            pltpu.VMEM((CHUNK,), jnp.int32),
            pltpu.VMEM((CHUNK, D_FEAT), jnp.float32),
            pltpu.VMEM((CHUNK, D_FEAT), jnp.float32),
            pltpu.VMEM((CHUNK,), jnp.float32),
            pltpu.SemaphoreType.DMA,
            pltpu.SemaphoreType.DMA,
        ],
    )
    def edge_kernel(h_hbm, src_hbm, dst_hbm, norms_hbm, out_hbm,
                    norms_v, sids_v, dids_v, srows_v, drows_v, outc_v,
                    sem_a, sem_b):
        wid = lax.axis_index("s") * num_cores + lax.axis_index("c")
        wbase = wid * e_per_w
        pltpu.sync_copy(norms_hbm, norms_v)

        def chunk_body(c, carry):
            cbase = wbase + c * CHUNK
            pltpu.sync_copy(src_hbm.at[pl.ds(cbase, CHUNK)], sids_v)
            pltpu.sync_copy(dst_hbm.at[pl.ds(cbase, CHUNK)], dids_v)
            cp1 = pltpu.async_copy(h_hbm.at[sids_v], srows_v, sem_a)
            cp2 = pltpu.async_copy(h_hbm.at[dids_v], drows_v, sem_b)
            cp1.wait()
            cp2.wait()

            def group_body(g, gcarry):
                rows = g * 16 + lax.iota(jnp.int32, 16)

                def dot_body(dd, acc):
                    col = jnp.broadcast_to(dd, (16,))
                    s = plsc.load_gather(srows_v, [rows, col])
                    t = plsc.load_gather(drows_v, [rows, col])
                    return acc + s * t

                num = lax.fori_loop(0, D_FEAT, dot_body,
                                    jnp.zeros((16,), jnp.float32), unroll=8)
                sid = sids_v[pl.ds(g * 16, 16)]
                did = dids_v[pl.ds(g * 16, 16)]
                ns = plsc.load_gather(norms_v, [sid])
                nd = plsc.load_gather(norms_v, [did])
                denom = jnp.maximum(ns * nd, jnp.float32(1e-6))
                outc_v[pl.ds(g * 16, 16)] = num / denom
                return gcarry

            lax.fori_loop(0, GROUPS, group_body, 0)
            pltpu.sync_copy(outc_v, out_hbm.at[pl.ds(cbase, CHUNK)])
            return carry

        lax.fori_loop(0, n_chunks, chunk_body, 0)

    return edge_kernel


def kernel(h, edge_index):
    h = h.astype(jnp.float32)
    ei = edge_index.astype(jnp.int32)
    src = ei[0]
    dst = ei[1]
    norms = _node_norms(h)
    return _make_edge_kernel()(h, src, dst, norms)


# speedup vs baseline: 1.3971x; 1.2168x over previous
import functools

import jax
import jax.numpy as jnp
from jax import lax
from jax.experimental import pallas as pl
from jax.experimental.pallas import tpu as pltpu
from jax.experimental.pallas import tpu_sc as plsc

N_NODES = 10000
N_EDGES = 320000
D_FEAT = 128
CHUNK = 80
GROUPS = CHUNK // 16


def _norms_body(h_ref, out_ref):
    h = h_ref[...]
    out_ref[...] = jnp.sqrt(jnp.sum(h * h, axis=1))


def _node_norms(h):
    return pl.pallas_call(
        _norms_body,
        out_shape=jax.ShapeDtypeStruct((h.shape[0],), jnp.float32),
    )(h)


@functools.cache
def _make_edge_kernel():
    info = plsc.get_sparse_core_info()
    num_cores = info.num_cores
    nw = num_cores * info.num_subcores
    e_per_w = N_EDGES // nw
    n_chunks = e_per_w // CHUNK
    assert n_chunks % 2 == 1

    mesh = plsc.VectorSubcoreMesh(core_axis_name="c", subcore_axis_name="s")

    @functools.partial(
        pl.kernel,
        mesh=mesh,
        compiler_params=pltpu.CompilerParams(needs_layout_passes=False),
        out_type=jax.ShapeDtypeStruct((N_EDGES,), jnp.float32),
        scratch_types=[
            pltpu.VMEM((N_NODES,), jnp.float32),
            pltpu.VMEM((e_per_w,), jnp.int32),
            pltpu.VMEM((e_per_w,), jnp.int32),
            pltpu.VMEM((e_per_w,), jnp.float32),
            pltpu.VMEM((CHUNK, D_FEAT), jnp.float32),
            pltpu.VMEM((CHUNK, D_FEAT), jnp.float32),
            pltpu.VMEM((CHUNK, D_FEAT), jnp.float32),
            pltpu.VMEM((CHUNK, D_FEAT), jnp.float32),
            pltpu.SemaphoreType.DMA,
            pltpu.SemaphoreType.DMA,
        ],
    )
    def edge_kernel(h_hbm, src_hbm, dst_hbm, norms_hbm, out_hbm,
                    norms_v, sids_v, dids_v, out_v,
                    srows0, srows1, drows0, drows1,
                    sem0, sem1):
        wid = lax.axis_index("s") * num_cores + lax.axis_index("c")
        wbase = wid * e_per_w
        pltpu.sync_copy(src_hbm.at[pl.ds(wbase, e_per_w)], sids_v)
        pltpu.sync_copy(dst_hbm.at[pl.ds(wbase, e_per_w)], dids_v)
        pltpu.sync_copy(norms_hbm, norms_v)

        def start(c, sbuf, dbuf, sem):
            pltpu.async_copy(h_hbm.at[sids_v.at[pl.ds(c * CHUNK, CHUNK)]],
                             sbuf, sem)
            pltpu.async_copy(h_hbm.at[dids_v.at[pl.ds(c * CHUNK, CHUNK)]],
                             dbuf, sem)

        def drain(sbuf, dbuf, sem):
            pltpu.make_async_copy(h_hbm.at[pl.ds(0, CHUNK)], sbuf, sem).wait()
            pltpu.make_async_copy(h_hbm.at[pl.ds(0, CHUNK)], dbuf, sem).wait()

        def compute(c, sbuf, dbuf):
            def group_body(g, gcarry):
                rows = g * 16 + lax.iota(jnp.int32, 16)

                def dot_body(dd, acc):
                    col = jnp.broadcast_to(dd, (16,))
                    s = plsc.load_gather(sbuf, [rows, col])
                    t = plsc.load_gather(dbuf, [rows, col])
                    return acc + s * t

                num = lax.fori_loop(0, D_FEAT, dot_body,
                                    jnp.zeros((16,), jnp.float32), unroll=8)
                e0 = c * CHUNK + g * 16
                sid = sids_v[pl.ds(e0, 16)]
                did = dids_v[pl.ds(e0, 16)]
                ns = plsc.load_gather(norms_v, [sid])
                nd = plsc.load_gather(norms_v, [did])
                denom = jnp.maximum(ns * nd, jnp.float32(1e-6))
                out_v[pl.ds(e0, 16)] = num / denom
                return gcarry

            lax.fori_loop(0, GROUPS, group_body, 0)

        start(0, srows0, drows0, sem0)

        def pair_body(i, carry):
            c = i * 2
            start(c + 1, srows1, drows1, sem1)
            drain(srows0, drows0, sem0)
            compute(c, srows0, drows0)
            start(c + 2, srows0, drows0, sem0)
            drain(srows1, drows1, sem1)
            compute(c + 1, srows1, drows1)
            return carry

        lax.fori_loop(0, (n_chunks - 1) // 2, pair_body, 0)
        drain(srows0, drows0, sem0)
        compute(n_chunks - 1, srows0, drows0)

        pltpu.sync_copy(out_v, out_hbm.at[pl.ds(wbase, e_per_w)])

    return edge_kernel


def kernel(h, edge_index):
    h = h.astype(jnp.float32)
    ei = edge_index.astype(jnp.int32)
    src = ei[0]
    dst = ei[1]
    norms = _node_norms(h)
    return _make_edge_kernel()(h, src, dst, norms)


# speedup vs baseline: 1.5887x; 1.1371x over previous
import functools

import jax
import jax.numpy as jnp
from jax import lax
from jax.experimental import pallas as pl
from jax.experimental.pallas import tpu as pltpu
from jax.experimental.pallas import tpu_sc as plsc

N_NODES = 10000
N_EDGES = 320000
D_FEAT = 128
CHUNK = 80
GROUPS = CHUNK // 16


def _norms_body(h_ref, out_ref):
    h = h_ref[...]
    out_ref[...] = jnp.sqrt(jnp.sum(h * h, axis=1))


def _node_norms(h):
    return pl.pallas_call(
        _norms_body,
        out_shape=jax.ShapeDtypeStruct((h.shape[0],), jnp.float32),
    )(h)


@functools.cache
def _make_edge_kernel():
    info = plsc.get_sparse_core_info()
    num_cores = info.num_cores
    nw = num_cores * info.num_subcores
    e_per_w = N_EDGES // nw
    n_chunks = e_per_w // CHUNK
    assert n_chunks % 2 == 1

    mesh = plsc.VectorSubcoreMesh(core_axis_name="c", subcore_axis_name="s")

    @functools.partial(
        pl.kernel,
        mesh=mesh,
        compiler_params=pltpu.CompilerParams(needs_layout_passes=False),
        out_type=jax.ShapeDtypeStruct((N_EDGES,), jnp.float32),
        scratch_types=[
            pltpu.VMEM((N_NODES,), jnp.float32),
            pltpu.VMEM((e_per_w,), jnp.int32),
            pltpu.VMEM((e_per_w,), jnp.int32),
            pltpu.VMEM((e_per_w,), jnp.float32),
            pltpu.VMEM((CHUNK, D_FEAT), jnp.float32),
            pltpu.VMEM((CHUNK, D_FEAT), jnp.float32),
            pltpu.VMEM((CHUNK, D_FEAT), jnp.float32),
            pltpu.VMEM((CHUNK, D_FEAT), jnp.float32),
            pltpu.SemaphoreType.DMA,
            pltpu.SemaphoreType.DMA,
        ],
    )
    def edge_kernel(h_hbm, src_hbm, dst_hbm, norms_hbm, out_hbm,
                    norms_v, sids_v, dids_v, out_v,
                    srows0, srows1, drows0, drows1,
                    sem0, sem1):
        wid = lax.axis_index("s") * num_cores + lax.axis_index("c")
        wbase = wid * e_per_w
        pltpu.sync_copy(src_hbm.at[pl.ds(wbase, e_per_w)], sids_v)
        pltpu.sync_copy(dst_hbm.at[pl.ds(wbase, e_per_w)], dids_v)
        pltpu.sync_copy(norms_hbm, norms_v)

        def start(c, sbuf, dbuf, sem):
            pltpu.async_copy(h_hbm.at[sids_v.at[pl.ds(c * CHUNK, CHUNK)]],
                             sbuf, sem)
            pltpu.async_copy(h_hbm.at[dids_v.at[pl.ds(c * CHUNK, CHUNK)]],
                             dbuf, sem)

        def drain(sbuf, dbuf, sem):
            pltpu.make_async_copy(h_hbm.at[pl.ds(0, CHUNK)], sbuf, sem).wait()
            pltpu.make_async_copy(h_hbm.at[pl.ds(0, CHUNK)], dbuf, sem).wait()

        def compute(c, sbuf, dbuf):
            def group_body(g, gcarry):
                rows = g * 16 + lax.iota(jnp.int32, 16)

                def dot_body(k, accs):
                    base = k * 8
                    new_accs = []
                    for j in range(8):
                        col = jnp.broadcast_to(base + j, (16,))
                        s = plsc.load_gather(sbuf, [rows, col])
                        t = plsc.load_gather(dbuf, [rows, col])
                        new_accs.append(accs[j] + s * t)
                    return tuple(new_accs)

                zero = jnp.zeros((16,), jnp.float32)
                accs = lax.fori_loop(0, D_FEAT // 8, dot_body, (zero,) * 8)
                num = ((accs[0] + accs[1]) + (accs[2] + accs[3])) + (
                    (accs[4] + accs[5]) + (accs[6] + accs[7]))
                e0 = c * CHUNK + g * 16
                sid = sids_v[pl.ds(e0, 16)]
                did = dids_v[pl.ds(e0, 16)]
                ns = plsc.load_gather(norms_v, [sid])
                nd = plsc.load_gather(norms_v, [did])
                denom = jnp.maximum(ns * nd, jnp.float32(1e-6))
                out_v[pl.ds(e0, 16)] = num / denom
                return gcarry

            lax.fori_loop(0, GROUPS, group_body, 0)

        start(0, srows0, drows0, sem0)

        def pair_body(i, carry):
            c = i * 2
            start(c + 1, srows1, drows1, sem1)
            drain(srows0, drows0, sem0)
            compute(c, srows0, drows0)
            start(c + 2, srows0, drows0, sem0)
            drain(srows1, drows1, sem1)
            compute(c + 1, srows1, drows1)
            return carry

        lax.fori_loop(0, (n_chunks - 1) // 2, pair_body, 0)
        drain(srows0, drows0, sem0)
        compute(n_chunks - 1, srows0, drows0)

        pltpu.sync_copy(out_v, out_hbm.at[pl.ds(wbase, e_per_w)])

    return edge_kernel


def kernel(h, edge_index):
    h = h.astype(jnp.float32)
    ei = edge_index.astype(jnp.int32)
    src = ei[0]
    dst = ei[1]
    norms = _node_norms(h)
    return _make_edge_kernel()(h, src, dst, norms)


# speedup vs baseline: 4.0514x; 2.5502x over previous
import functools

import jax
import jax.numpy as jnp
from jax import lax
from jax.experimental import pallas as pl
from jax.experimental.pallas import tpu as pltpu
from jax.experimental.pallas import tpu_sc as plsc

N_NODES = 10000
N_EDGES = 320000
D_FEAT = 128
CHUNK = 80
GROUPS = CHUNK // 16


def _norms_body(h_ref, out_ref):
    h = h_ref[...]
    out_ref[...] = jnp.sqrt(jnp.sum(h * h, axis=1))


def _node_norms(h):
    return pl.pallas_call(
        _norms_body,
        out_shape=jax.ShapeDtypeStruct((h.shape[0],), jnp.float32),
    )(h)


@functools.cache
def _make_edge_kernel():
    info = plsc.get_sparse_core_info()
    num_cores = info.num_cores
    nw = num_cores * info.num_subcores
    e_per_w = N_EDGES // nw
    n_chunks = e_per_w // CHUNK
    assert n_chunks % 2 == 1

    mesh = plsc.VectorSubcoreMesh(core_axis_name="c", subcore_axis_name="s")

    @functools.partial(
        pl.kernel,
        mesh=mesh,
        compiler_params=pltpu.CompilerParams(needs_layout_passes=False),
        out_type=jax.ShapeDtypeStruct((N_EDGES,), jnp.float32),
        scratch_types=[
            pltpu.VMEM((N_NODES,), jnp.float32),
            pltpu.VMEM((e_per_w,), jnp.int32),
            pltpu.VMEM((e_per_w,), jnp.int32),
            pltpu.VMEM((e_per_w,), jnp.float32),
            pltpu.VMEM((CHUNK, D_FEAT), jnp.float32),
            pltpu.VMEM((CHUNK, D_FEAT), jnp.float32),
            pltpu.VMEM((CHUNK, D_FEAT), jnp.float32),
            pltpu.VMEM((CHUNK, D_FEAT), jnp.float32),
            pltpu.SemaphoreType.DMA,
            pltpu.SemaphoreType.DMA,
        ],
    )
    def edge_kernel(h_hbm, src_hbm, dst_hbm, norms_hbm, out_hbm,
                    norms_v, sids_v, dids_v, out_v,
                    srows0, srows1, drows0, drows1,
                    sem0, sem1):
        wid = lax.axis_index("s") * num_cores + lax.axis_index("c")
        wbase = wid * e_per_w
        pltpu.sync_copy(src_hbm.at[pl.ds(wbase, e_per_w)], sids_v)
        pltpu.sync_copy(dst_hbm.at[pl.ds(wbase, e_per_w)], dids_v)
        pltpu.sync_copy(norms_hbm, norms_v)

        def start(c, sbuf, dbuf, sem):
            pltpu.async_copy(h_hbm.at[sids_v.at[pl.ds(c * CHUNK, CHUNK)]],
                             sbuf, sem)
            pltpu.async_copy(h_hbm.at[dids_v.at[pl.ds(c * CHUNK, CHUNK)]],
                             dbuf, sem)

        def drain(sbuf, dbuf, sem):
            pltpu.make_async_copy(h_hbm.at[pl.ds(0, CHUNK)], sbuf, sem).wait()
            pltpu.make_async_copy(h_hbm.at[pl.ds(0, CHUNK)], dbuf, sem).wait()

        lane = lax.iota(jnp.int32, 16)
        perms = [lane ^ step for step in (8, 4, 2, 1)]
        masks = [lane == e for e in range(16)]
        zero = jnp.zeros((16,), jnp.float32)

        def compute(c, sbuf, dbuf):
            def group_body(g, gcarry):
                num_vec = zero
                for e in range(16):
                    row = g * 16 + e
                    prods = []
                    for k in range(8):
                        s = sbuf[row, pl.ds(k * 16, 16)]
                        t = dbuf[row, pl.ds(k * 16, 16)]
                        prods.append(s * t)
                    acc = ((prods[0] + prods[1]) + (prods[2] + prods[3])) + (
                        (prods[4] + prods[5]) + (prods[6] + prods[7]))
                    for p in perms:
                        acc = acc + acc.at[p].get(mode="promise_in_bounds")
                    num_vec = jnp.where(masks[e], acc, num_vec)
                e0 = c * CHUNK + g * 16
                sid = sids_v[pl.ds(e0, 16)]
                did = dids_v[pl.ds(e0, 16)]
                ns = plsc.load_gather(norms_v, [sid])
                nd = plsc.load_gather(norms_v, [did])
                denom = jnp.maximum(ns * nd, jnp.float32(1e-6))
                out_v[pl.ds(e0, 16)] = num_vec / denom
                return gcarry

            lax.fori_loop(0, GROUPS, group_body, 0)

        start(0, srows0, drows0, sem0)

        def pair_body(i, carry):
            c = i * 2
            start(c + 1, srows1, drows1, sem1)
            drain(srows0, drows0, sem0)
            compute(c, srows0, drows0)
            start(c + 2, srows0, drows0, sem0)
            drain(srows1, drows1, sem1)
            compute(c + 1, srows1, drows1)
            return carry

        lax.fori_loop(0, (n_chunks - 1) // 2, pair_body, 0)
        drain(srows0, drows0, sem0)
        compute(n_chunks - 1, srows0, drows0)

        pltpu.sync_copy(out_v, out_hbm.at[pl.ds(wbase, e_per_w)])

    return edge_kernel


def kernel(h, edge_index):
    h = h.astype(jnp.float32)
    ei = edge_index.astype(jnp.int32)
    src = ei[0]
    dst = ei[1]
    norms = _node_norms(h)
    return _make_edge_kernel()(h, src, dst, norms)


# speedup vs baseline: 8.9774x; 2.2159x over previous
import functools

import jax
import jax.numpy as jnp
from jax import lax
from jax.experimental import pallas as pl
from jax.experimental.pallas import tpu as pltpu
from jax.experimental.pallas import tpu_sc as plsc

N_NODES = 10000
N_EDGES = 320000
D_FEAT = 128
CHUNK = 80
GROUPS = CHUNK // 16


def _norms_body(h_ref, out_ref):
    h = h_ref[...]
    out_ref[...] = jnp.sqrt(jnp.sum(h * h, axis=1))


def _node_norms(h):
    return pl.pallas_call(
        _norms_body,
        out_shape=jax.ShapeDtypeStruct((h.shape[0],), jnp.float32),
    )(h)


@functools.cache
def _make_edge_kernel():
    info = plsc.get_sparse_core_info()
    num_cores = info.num_cores
    nw = num_cores * info.num_subcores
    e_per_w = N_EDGES // nw
    n_chunks = e_per_w // CHUNK
    assert n_chunks % 2 == 1

    mesh = plsc.VectorSubcoreMesh(core_axis_name="c", subcore_axis_name="s")

    @functools.partial(
        pl.kernel,
        mesh=mesh,
        compiler_params=pltpu.CompilerParams(needs_layout_passes=False),
        out_type=jax.ShapeDtypeStruct((N_EDGES,), jnp.float32),
        scratch_types=[
            pltpu.VMEM((N_NODES,), jnp.float32),
            pltpu.VMEM((e_per_w,), jnp.int32),
            pltpu.VMEM((e_per_w,), jnp.int32),
            pltpu.VMEM((e_per_w,), jnp.float32),
            pltpu.VMEM((CHUNK, D_FEAT), jnp.float32),
            pltpu.VMEM((CHUNK, D_FEAT), jnp.float32),
            pltpu.VMEM((CHUNK, D_FEAT), jnp.float32),
            pltpu.VMEM((CHUNK, D_FEAT), jnp.float32),
            pltpu.SemaphoreType.DMA,
            pltpu.SemaphoreType.DMA,
        ],
    )
    def edge_kernel(h_hbm, src_hbm, dst_hbm, norms_hbm, out_hbm,
                    norms_v, sids_v, dids_v, out_v,
                    srows0, srows1, drows0, drows1,
                    sem0, sem1):
        wid = lax.axis_index("s") * num_cores + lax.axis_index("c")
        wbase = wid * e_per_w
        pltpu.sync_copy(src_hbm.at[pl.ds(wbase, e_per_w)], sids_v)
        pltpu.sync_copy(dst_hbm.at[pl.ds(wbase, e_per_w)], dids_v)
        pltpu.sync_copy(norms_hbm, norms_v)

        def start(c, sbuf, dbuf, sem):
            pltpu.async_copy(h_hbm.at[sids_v.at[pl.ds(c * CHUNK, CHUNK)]],
                             sbuf, sem)
            pltpu.async_copy(h_hbm.at[dids_v.at[pl.ds(c * CHUNK, CHUNK)]],
                             dbuf, sem)

        def drain(sbuf, dbuf, sem):
            pltpu.make_async_copy(h_hbm.at[pl.ds(0, CHUNK)], sbuf, sem).wait()
            pltpu.make_async_copy(h_hbm.at[pl.ds(0, CHUNK)], dbuf, sem).wait()

        lane = lax.iota(jnp.int32, 16)
        perms = [lane ^ step for step in (8, 4, 2, 1)]
        zero = jnp.zeros((16,), jnp.float32)

        def compute(c, sbuf, dbuf):
            def group_body(g, gcarry):
                def edge_body(e, num_acc):
                    row = g * 16 + e
                    prods = []
                    for k in range(8):
                        s = sbuf[row, pl.ds(k * 16, 16)]
                        t = dbuf[row, pl.ds(k * 16, 16)]
                        prods.append(s * t)
                    acc = ((prods[0] + prods[1]) + (prods[2] + prods[3])) + (
                        (prods[4] + prods[5]) + (prods[6] + prods[7]))
                    for p in perms:
                        acc = acc + acc.at[p].get(mode="promise_in_bounds")
                    m = lane == jnp.broadcast_to(e, (16,))
                    return jnp.where(m, acc, num_acc)

                num_vec = plsc.parallel_loop(0, 16, carry=zero,
                                             unroll=2)(edge_body)
                e0 = c * CHUNK + g * 16
                sid = sids_v[pl.ds(e0, 16)]
                did = dids_v[pl.ds(e0, 16)]
                ns = plsc.load_gather(norms_v, [sid])
                nd = plsc.load_gather(norms_v, [did])
                denom = jnp.maximum(ns * nd, jnp.float32(1e-6))
                out_v[pl.ds(e0, 16)] = num_vec / denom
                return gcarry

            lax.fori_loop(0, GROUPS, group_body, 0)

        start(0, srows0, drows0, sem0)

        def pair_body(i, carry):
            c = i * 2
            start(c + 1, srows1, drows1, sem1)
            drain(srows0, drows0, sem0)
            compute(c, srows0, drows0)
            start(c + 2, srows0, drows0, sem0)
            drain(srows1, drows1, sem1)
            compute(c + 1, srows1, drows1)
            return carry

        lax.fori_loop(0, (n_chunks - 1) // 2, pair_body, 0)
        drain(srows0, drows0, sem0)
        compute(n_chunks - 1, srows0, drows0)

        pltpu.sync_copy(out_v, out_hbm.at[pl.ds(wbase, e_per_w)])

    return edge_kernel


def kernel(h, edge_index):
    h = h.astype(jnp.float32)
    ei = edge_index.astype(jnp.int32)
    src = ei[0]
    dst = ei[1]
    norms = _node_norms(h)
    return _make_edge_kernel()(h, src, dst, norms)
